# baseline (speedup 1.0000x reference)
import functools

import jax
import jax.numpy as jnp
from jax import lax
from jax.experimental import pallas as pl
from jax.experimental.pallas import tpu as pltpu
from jax.experimental.pallas import tpu_sc as plsc

N = 1024
V = 100000
SRC_L = 200
PAD_ID = 0
EPS = 1e-12

RB1 = 128
VB = 2048
NRB = N // RB1
NVB = -(-V // VB)
TAIL = V - (NVB - 1) * VB

SC_NC = 2
SC_NS = 16
SC_NW = SC_NC * SC_NS
SC_CHUNK = 128
SC_CPW = 56
SC_GROUP = 7
SC_TOTAL = SC_NW * SC_CPW * SC_CHUNK
NIDX = N * SRC_L + N

BIGI = 2**30


def _stage1_body(x_ref, m_ref, bb_ref):
    vb = pl.program_id(1)

    @pl.when(vb == 0)
    def _init():
        m_ref[0, 0, :] = jnp.full((RB1,), -1.0, jnp.float32)
        bb_ref[0, 0, :] = jnp.zeros((RB1,), jnp.int32)

    def merge(x):
        bm = jnp.max(x, axis=1)
        m_old = m_ref[0, 0, :]
        better = bm > m_old
        m_ref[0, 0, :] = jnp.where(better, bm, m_old)
        bb_ref[0, 0, :] = jnp.where(better, vb, bb_ref[0, 0, :])

    @pl.when(vb < NVB - 1)
    def _full():
        merge(x_ref[...])

    @pl.when(vb == NVB - 1)
    def _tail():
        col = lax.broadcasted_iota(jnp.int32, (RB1, VB), 1)
        merge(jnp.where(col < TAIL, x_ref[...], -1.0))


def _run_stage1(p_gen):
    return pl.pallas_call(
        _stage1_body,
        grid=(NRB, NVB),
        in_specs=[pl.BlockSpec((RB1, VB), lambda rb, vb: (rb, vb))],
        out_specs=[
            pl.BlockSpec((1, 1, RB1), lambda rb, vb: (rb, 0, 0)),
            pl.BlockSpec((1, 1, RB1), lambda rb, vb: (rb, 0, 0)),
        ],
        out_shape=[
            jax.ShapeDtypeStruct((NRB, 1, RB1), jnp.float32),
            jax.ShapeDtypeStruct((NRB, 1, RB1), jnp.int32),
        ],
    )(p_gen)


def _sc_gather_body(flat_hbm, idx_hbm, out_hbm, idx_v, rows_v, sem):
    wid = lax.axis_index("s") * SC_NC + lax.axis_index("c")
    base = wid * SC_CPW
    pltpu.sync_copy(idx_hbm.at[pl.ds(base, SC_CPW)], idx_v)

    def group(g, _):
        copies = []
        for j in range(SC_GROUP):
            c = g * SC_GROUP + j
            cp = pltpu.make_async_copy(
                flat_hbm.at[idx_v.at[c]], rows_v.at[c], sem)
            cp.start()
            copies.append(cp)
        for cp in copies:
            cp.wait()
        return 0

    lax.fori_loop(0, SC_CPW // SC_GROUP, group, 0)
    pltpu.sync_copy(rows_v, out_hbm.at[pl.ds(base, SC_CPW)])


def _run_sc_gather(flat_pgen, idx2d):
    mesh = plsc.VectorSubcoreMesh(core_axis_name="c", subcore_axis_name="s")
    fn = functools.partial(
        pl.kernel,
        out_type=jax.ShapeDtypeStruct((SC_NW * SC_CPW, SC_CHUNK), jnp.float32),
        mesh=mesh,
        scratch_types=[
            pltpu.VMEM((SC_CPW, SC_CHUNK), jnp.int32),
            pltpu.VMEM((SC_CPW, SC_CHUNK), jnp.float32),
            pltpu.SemaphoreType.DMA,
        ],
    )(_sc_gather_body)
    return fn(flat_pgen, idx2d)


WROW = 512
WIN_ROWS = 5
WIN_W = WROW * WIN_ROWS
NTROW = N * V // WROW
K0_MAX = NTROW - WIN_ROWS
WG_PW = N * WIN_ROWS // SC_NW
WG_HALF = WG_PW // 2


def _win_gather_body(tab_hbm, idx_hbm, out_hbm, idx_v, rows_v, sem):
    wid = lax.axis_index("s") * SC_NC + lax.axis_index("c")
    base = wid * WG_PW
    pltpu.sync_copy(idx_hbm.at[pl.ds(base, WG_PW)], idx_v)
    c0 = pltpu.make_async_copy(
        tab_hbm.at[idx_v.at[pl.ds(0, WG_HALF)]],
        rows_v.at[pl.ds(0, WG_HALF)], sem)
    c0.start()
    c1 = pltpu.make_async_copy(
        tab_hbm.at[idx_v.at[pl.ds(WG_HALF, WG_HALF)]],
        rows_v.at[pl.ds(WG_HALF, WG_HALF)], sem)
    c1.start()
    c0.wait()
    c1.wait()
    pltpu.sync_copy(rows_v, out_hbm.at[pl.ds(base, WG_PW)])


def _run_win_gather(flat_pgen, widx):
    mesh = plsc.VectorSubcoreMesh(core_axis_name="c", subcore_axis_name="s")
    fn = functools.partial(
        pl.kernel,
        out_type=jax.ShapeDtypeStruct((N * WIN_ROWS, WROW), jnp.float32),
        mesh=mesh,
        scratch_types=[
            pltpu.VMEM((WG_PW,), jnp.int32),
            pltpu.VMEM((WG_PW, WROW), jnp.float32),
            pltpu.SemaphoreType.DMA,
        ],
    )(_win_gather_body)
    return fn(flat_pgen.reshape(NTROW, WROW), widx)


def _stage3_body(win_ref, cb_ref, m0_ref, a0_ref):
    win = win_ref[...]
    cb = cb_ref[0, 0, :]
    m0 = m0_ref[0, 0, :]
    col = cb[:, None] + lax.broadcasted_iota(jnp.int32, (RB1, WIN_W), 1)
    hit = jnp.logical_and(
        jnp.logical_and(col >= 0, col < V), win == m0[:, None])
    a0_ref[0, 0, :] = jnp.min(jnp.where(hit, col, BIGI), axis=1)


def _run_stage3(win2d, cb3d, m03d):
    return pl.pallas_call(
        _stage3_body,
        grid=(NRB,),
        in_specs=[
            pl.BlockSpec((RB1, WIN_W), lambda i: (i, 0)),
            pl.BlockSpec((1, 1, RB1), lambda i: (i, 0, 0)),
            pl.BlockSpec((1, 1, RB1), lambda i: (i, 0, 0)),
        ],
        out_specs=[pl.BlockSpec((1, 1, RB1), lambda i: (i, 0, 0))],
        out_shape=[jax.ShapeDtypeStruct((NRB, 1, RB1), jnp.int32)],
    )(win2d, cb3d, m03d)[0]


DUP_CH = 8


def _stage4_body(pc_ref, src_ref, pgs_ref, s_ref, tgt_ref, m0_ref, a0_ref,
                 gt_ref, loss_ref, pred_ref, nw_ref, nc_ref):
    step = pl.program_id(0)
    pc = pc_ref[...]
    src = src_ref[...]
    pgs = pgs_ref[...]
    s = s_ref[0, 0, :]
    tgt = tgt_ref[0, 0, :]
    m0 = m0_ref[0, 0, :]
    a0 = a0_ref[0, 0, :]
    g_t = gt_ref[0, 0, :]

    acc = jnp.zeros((RB1, SRC_L), jnp.float32)
    for c in range(SRC_L // DUP_CH):
        srcc = src[:, c * DUP_CH:(c + 1) * DUP_CH]
        pcc = pc[:, c * DUP_CH:(c + 1) * DUP_CH]
        eq = srcc[:, :, None] == src[:, None, :]
        acc = acc + jnp.sum(jnp.where(eq, pcc[:, :, None], 0.0), axis=1)

    one_m_s = 1.0 - s
    adj = one_m_s[:, None] * pgs + s[:, None] * acc
    base = one_m_s * m0

    adjmax = jnp.max(adj, axis=1)
    eqa = adj == adjmax[:, None]
    adjidx = jnp.min(jnp.where(eqa, src, BIGI), axis=1)
    pred = jnp.where(
        adjmax > base, adjidx,
        jnp.where(adjmax == base, jnp.minimum(adjidx, a0), a0))
    pred_ref[0, 0, :] = pred

    copy_t = jnp.sum(jnp.where(src == tgt[:, None], pc, 0.0), axis=1)
    tscore = one_m_s * g_t + s * copy_t
    nonpad = tgt != PAD_ID
    nonpad_f = nonpad.astype(jnp.float32)
    loss_part = jnp.sum(jnp.log(tscore + EPS) * nonpad_f) * (-1.0 / N)
    nw_part = jnp.sum(nonpad.astype(jnp.int32))
    nc_part = jnp.sum(jnp.logical_and(pred == tgt, nonpad).astype(jnp.int32))

    @pl.when(step == 0)
    def _init():
        loss_ref[0, 0] = 0.0
        nw_ref[0, 0] = 0
        nc_ref[0, 0] = 0

    loss_ref[0, 0] += loss_part
    nw_ref[0, 0] += nw_part
    nc_ref[0, 0] += nc_part


def _run_stage4(p_copy, src, pg_src, s3d, tgt3d, m03d, a03d, gt3d):
    return pl.pallas_call(
        _stage4_body,
        grid=(NRB,),
        in_specs=[
            pl.BlockSpec((RB1, SRC_L), lambda i: (i, 0)),
            pl.BlockSpec((RB1, SRC_L), lambda i: (i, 0)),
            pl.BlockSpec((RB1, SRC_L), lambda i: (i, 0)),
            pl.BlockSpec((1, 1, RB1), lambda i: (i, 0, 0)),
            pl.BlockSpec((1, 1, RB1), lambda i: (i, 0, 0)),
            pl.BlockSpec((1, 1, RB1), lambda i: (i, 0, 0)),
            pl.BlockSpec((1, 1, RB1), lambda i: (i, 0, 0)),
            pl.BlockSpec((1, 1, RB1), lambda i: (i, 0, 0)),
        ],
        out_specs=[
            pl.BlockSpec(memory_space=pltpu.SMEM, block_shape=(1, 1),
                         index_map=lambda i: (0, 0)),
            pl.BlockSpec((1, 1, RB1), lambda i: (i, 0, 0)),
            pl.BlockSpec(memory_space=pltpu.SMEM, block_shape=(1, 1),
                         index_map=lambda i: (0, 0)),
            pl.BlockSpec(memory_space=pltpu.SMEM, block_shape=(1, 1),
                         index_map=lambda i: (0, 0)),
        ],
        out_shape=[
            jax.ShapeDtypeStruct((1, 1), jnp.float32),
            jax.ShapeDtypeStruct((NRB, 1, RB1), jnp.int32),
            jax.ShapeDtypeStruct((1, 1), jnp.int32),
            jax.ShapeDtypeStruct((1, 1), jnp.int32),
        ],
    )(p_copy, src, pg_src, s3d, tgt3d, m03d, a03d, gt3d)


def kernel(p_gen, p_copy, p_switch, tgt, src):
    tgt = tgt.astype(jnp.int32)
    src = src.astype(jnp.int32)

    m03d, bb3d = _run_stage1(p_gen)

    rows = jnp.arange(N, dtype=jnp.int32) * V
    flat_idx = jnp.concatenate([
        (rows[:, None] + src).reshape(-1),
        rows + tgt,
        jnp.zeros((SC_TOTAL - NIDX,), jnp.int32),
    ])
    flat_pgen = p_gen.reshape(-1)
    gathered = _run_sc_gather(
        flat_pgen, flat_idx.reshape(SC_NW * SC_CPW, SC_CHUNK))
    gflat = gathered.reshape(-1)
    pg_src = gflat[:N * SRC_L].reshape(N, SRC_L)
    g_t = gflat[N * SRC_L:NIDX]

    bb = bb3d.reshape(N)
    rows_iv = jnp.arange(N, dtype=jnp.int32) * V
    k0 = jnp.minimum((rows_iv + bb * VB) // WROW, K0_MAX)
    widx = (k0[:, None] + jnp.arange(WIN_ROWS, dtype=jnp.int32)).reshape(-1)
    win = _run_win_gather(flat_pgen, widx).reshape(N, WIN_W)
    cb3d = (k0 * WROW - rows_iv).reshape(NRB, 1, RB1)
    a03d = _run_stage3(win, cb3d, m03d)

    loss2d, pred3d, nw2d, nc2d = _run_stage4(
        p_copy, src, pg_src,
        p_switch.reshape(NRB, 1, RB1), tgt.reshape(NRB, 1, RB1),
        m03d, a03d, g_t.reshape(NRB, 1, RB1))

    return (loss2d[0, 0], pred3d.reshape(N), nw2d[0, 0], nc2d[0, 0])


# speedup vs baseline: 1.0634x; 1.0634x over previous
import functools

import jax
import jax.numpy as jnp
from jax import lax
from jax.experimental import pallas as pl
from jax.experimental.pallas import tpu as pltpu
from jax.experimental.pallas import tpu_sc as plsc

N = 1024
V = 100000
SRC_L = 200
PAD_ID = 0
EPS = 1e-12

RB1 = 128
VB = 8192
BLK = 2048
SUBS = VB // BLK
NRB = N // RB1
NVB = -(-V // VB)
TAIL = V - (NVB - 1) * VB

SC_NC = 2
SC_NS = 16
SC_NW = SC_NC * SC_NS
SC_CHUNK = 128
SC_CPW = 56
SC_GROUP = 7
SC_TOTAL = SC_NW * SC_CPW * SC_CHUNK
NIDX = N * SRC_L + N

BIGI = 2**30


def _stage1_body(x_ref, m_ref, bb_ref):
    vb = pl.program_id(1)

    @pl.when(vb == 0)
    def _init():
        m_ref[0, 0, :] = jnp.full((RB1,), -1.0, jnp.float32)
        bb_ref[0, 0, :] = jnp.zeros((RB1,), jnp.int32)

    def merge(x):
        for sub in range(SUBS):
            bm = jnp.max(x[:, sub * BLK:(sub + 1) * BLK], axis=1)
            m_old = m_ref[0, 0, :]
            better = bm > m_old
            m_ref[0, 0, :] = jnp.where(better, bm, m_old)
            bb_ref[0, 0, :] = jnp.where(
                better, vb * SUBS + sub, bb_ref[0, 0, :])

    @pl.when(vb < NVB - 1)
    def _full():
        merge(x_ref[...])

    @pl.when(vb == NVB - 1)
    def _tail():
        col = lax.broadcasted_iota(jnp.int32, (RB1, VB), 1)
        merge(jnp.where(col < TAIL, x_ref[...], -1.0))


def _run_stage1(p_gen):
    return pl.pallas_call(
        _stage1_body,
        grid=(NRB, NVB),
        in_specs=[pl.BlockSpec((RB1, VB), lambda rb, vb: (rb, vb))],
        out_specs=[
            pl.BlockSpec((1, 1, RB1), lambda rb, vb: (rb, 0, 0)),
            pl.BlockSpec((1, 1, RB1), lambda rb, vb: (rb, 0, 0)),
        ],
        out_shape=[
            jax.ShapeDtypeStruct((NRB, 1, RB1), jnp.float32),
            jax.ShapeDtypeStruct((NRB, 1, RB1), jnp.int32),
        ],
        compiler_params=pltpu.CompilerParams(
            dimension_semantics=(pltpu.PARALLEL, pltpu.ARBITRARY)),
    )(p_gen)


def _sc_gather_body(flat_hbm, idx_hbm, out_hbm, idx_v, rows_v, sem):
    wid = lax.axis_index("s") * SC_NC + lax.axis_index("c")
    base = wid * SC_CPW
    pltpu.sync_copy(idx_hbm.at[pl.ds(base, SC_CPW)], idx_v)

    def fire(g, _):
        for j in range(SC_GROUP):
            c = g * SC_GROUP + j
            pltpu.make_async_copy(
                flat_hbm.at[idx_v.at[c]], rows_v.at[c], sem).start()
        return 0

    def drain(g, _):
        for j in range(SC_GROUP):
            c = g * SC_GROUP + j
            pltpu.make_async_copy(
                flat_hbm.at[idx_v.at[c]], rows_v.at[c], sem).wait()
        return 0

    lax.fori_loop(0, SC_CPW // SC_GROUP, fire, 0)
    lax.fori_loop(0, SC_CPW // SC_GROUP, drain, 0)
    pltpu.sync_copy(rows_v, out_hbm.at[pl.ds(base, SC_CPW)])


def _run_sc_gather(flat_pgen, idx2d):
    mesh = plsc.VectorSubcoreMesh(core_axis_name="c", subcore_axis_name="s")
    fn = functools.partial(
        pl.kernel,
        out_type=jax.ShapeDtypeStruct((SC_NW * SC_CPW, SC_CHUNK), jnp.float32),
        mesh=mesh,
        scratch_types=[
            pltpu.VMEM((SC_CPW, SC_CHUNK), jnp.int32),
            pltpu.VMEM((SC_CPW, SC_CHUNK), jnp.float32),
            pltpu.SemaphoreType.DMA,
        ],
    )(_sc_gather_body)
    return fn(flat_pgen, idx2d)


WROW = 512
WIN_ROWS = 5
WIN_W = WROW * WIN_ROWS
NTROW = N * V // WROW
K0_MAX = NTROW - WIN_ROWS
WG_PW = N * WIN_ROWS // SC_NW
WG_HALF = WG_PW // 2


def _win_gather_body(tab_hbm, idx_hbm, out_hbm, idx_v, rows_v, sem):
    wid = lax.axis_index("s") * SC_NC + lax.axis_index("c")
    base = wid * WG_PW
    pltpu.sync_copy(idx_hbm.at[pl.ds(base, WG_PW)], idx_v)
    c0 = pltpu.make_async_copy(
        tab_hbm.at[idx_v.at[pl.ds(0, WG_HALF)]],
        rows_v.at[pl.ds(0, WG_HALF)], sem)
    c0.start()
    c1 = pltpu.make_async_copy(
        tab_hbm.at[idx_v.at[pl.ds(WG_HALF, WG_HALF)]],
        rows_v.at[pl.ds(WG_HALF, WG_HALF)], sem)
    c1.start()
    c0.wait()
    c1.wait()
    pltpu.sync_copy(rows_v, out_hbm.at[pl.ds(base, WG_PW)])


def _run_win_gather(flat_pgen, widx):
    mesh = plsc.VectorSubcoreMesh(core_axis_name="c", subcore_axis_name="s")
    fn = functools.partial(
        pl.kernel,
        out_type=jax.ShapeDtypeStruct((N * WIN_ROWS, WROW), jnp.float32),
        mesh=mesh,
        scratch_types=[
            pltpu.VMEM((WG_PW,), jnp.int32),
            pltpu.VMEM((WG_PW, WROW), jnp.float32),
            pltpu.SemaphoreType.DMA,
        ],
    )(_win_gather_body)
    return fn(flat_pgen.reshape(NTROW, WROW), widx)


def _stage3_body(win_ref, cb_ref, m0_ref, a0_ref):
    win = win_ref[...]
    cb = cb_ref[0, 0, :]
    m0 = m0_ref[0, 0, :]
    col = cb[:, None] + lax.broadcasted_iota(jnp.int32, (RB1, WIN_W), 1)
    hit = jnp.logical_and(
        jnp.logical_and(col >= 0, col < V), win == m0[:, None])
    a0_ref[0, 0, :] = jnp.min(jnp.where(hit, col, BIGI), axis=1)


def _run_stage3(win2d, cb3d, m03d):
    return pl.pallas_call(
        _stage3_body,
        grid=(NRB,),
        in_specs=[
            pl.BlockSpec((RB1, WIN_W), lambda i: (i, 0)),
            pl.BlockSpec((1, 1, RB1), lambda i: (i, 0, 0)),
            pl.BlockSpec((1, 1, RB1), lambda i: (i, 0, 0)),
        ],
        out_specs=[pl.BlockSpec((1, 1, RB1), lambda i: (i, 0, 0))],
        out_shape=[jax.ShapeDtypeStruct((NRB, 1, RB1), jnp.int32)],
    )(win2d, cb3d, m03d)[0]


DUP_CH = 40


def _stage4_body(pc_ref, src_ref, pgs_ref, s_ref, tgt_ref, m0_ref, a0_ref,
                 gt_ref, loss_ref, pred_ref, nw_ref, nc_ref):
    step = pl.program_id(0)
    pc = pc_ref[...]
    src = src_ref[...]
    pgs = pgs_ref[...]
    s = s_ref[0, 0, :]
    tgt = tgt_ref[0, 0, :]
    m0 = m0_ref[0, 0, :]
    a0 = a0_ref[0, 0, :]
    g_t = gt_ref[0, 0, :]

    acc = jnp.zeros((RB1, SRC_L), jnp.float32)
    for c in range(SRC_L // DUP_CH):
        srcc = src[:, c * DUP_CH:(c + 1) * DUP_CH]
        pcc = pc[:, c * DUP_CH:(c + 1) * DUP_CH]
        eq = srcc[:, :, None] == src[:, None, :]
        acc = acc + jnp.sum(jnp.where(eq, pcc[:, :, None], 0.0), axis=1)

    one_m_s = 1.0 - s
    adj = one_m_s[:, None] * pgs + s[:, None] * acc
    base = one_m_s * m0

    adjmax = jnp.max(adj, axis=1)
    eqa = adj == adjmax[:, None]
    adjidx = jnp.min(jnp.where(eqa, src, BIGI), axis=1)
    pred = jnp.where(
        adjmax > base, adjidx,
        jnp.where(adjmax == base, jnp.minimum(adjidx, a0), a0))
    pred_ref[0, 0, :] = pred

    copy_t = jnp.sum(jnp.where(src == tgt[:, None], pc, 0.0), axis=1)
    tscore = one_m_s * g_t + s * copy_t
    nonpad = tgt != PAD_ID
    nonpad_f = nonpad.astype(jnp.float32)
    loss_part = jnp.sum(jnp.log(tscore + EPS) * nonpad_f) * (-1.0 / N)
    nw_part = jnp.sum(nonpad.astype(jnp.int32))
    nc_part = jnp.sum(jnp.logical_and(pred == tgt, nonpad).astype(jnp.int32))

    @pl.when(step == 0)
    def _init():
        loss_ref[0, 0] = 0.0
        nw_ref[0, 0] = 0
        nc_ref[0, 0] = 0

    loss_ref[0, 0] += loss_part
    nw_ref[0, 0] += nw_part
    nc_ref[0, 0] += nc_part


def _run_stage4(p_copy, src, pg_src, s3d, tgt3d, m03d, a03d, gt3d):
    return pl.pallas_call(
        _stage4_body,
        grid=(NRB,),
        in_specs=[
            pl.BlockSpec((RB1, SRC_L), lambda i: (i, 0)),
            pl.BlockSpec((RB1, SRC_L), lambda i: (i, 0)),
            pl.BlockSpec((RB1, SRC_L), lambda i: (i, 0)),
            pl.BlockSpec((1, 1, RB1), lambda i: (i, 0, 0)),
            pl.BlockSpec((1, 1, RB1), lambda i: (i, 0, 0)),
            pl.BlockSpec((1, 1, RB1), lambda i: (i, 0, 0)),
            pl.BlockSpec((1, 1, RB1), lambda i: (i, 0, 0)),
            pl.BlockSpec((1, 1, RB1), lambda i: (i, 0, 0)),
        ],
        out_specs=[
            pl.BlockSpec(memory_space=pltpu.SMEM, block_shape=(1, 1),
                         index_map=lambda i: (0, 0)),
            pl.BlockSpec((1, 1, RB1), lambda i: (i, 0, 0)),
            pl.BlockSpec(memory_space=pltpu.SMEM, block_shape=(1, 1),
                         index_map=lambda i: (0, 0)),
            pl.BlockSpec(memory_space=pltpu.SMEM, block_shape=(1, 1),
                         index_map=lambda i: (0, 0)),
        ],
        out_shape=[
            jax.ShapeDtypeStruct((1, 1), jnp.float32),
            jax.ShapeDtypeStruct((NRB, 1, RB1), jnp.int32),
            jax.ShapeDtypeStruct((1, 1), jnp.int32),
            jax.ShapeDtypeStruct((1, 1), jnp.int32),
        ],
    )(p_copy, src, pg_src, s3d, tgt3d, m03d, a03d, gt3d)


def kernel(p_gen, p_copy, p_switch, tgt, src):
    tgt = tgt.astype(jnp.int32)
    src = src.astype(jnp.int32)

    m03d, bb3d = _run_stage1(p_gen)

    rows = jnp.arange(N, dtype=jnp.int32) * V
    flat_idx = jnp.concatenate([
        (rows[:, None] + src).reshape(-1),
        rows + tgt,
        jnp.zeros((SC_TOTAL - NIDX,), jnp.int32),
    ])
    flat_pgen = p_gen.reshape(-1)
    gathered = _run_sc_gather(
        flat_pgen, flat_idx.reshape(SC_NW * SC_CPW, SC_CHUNK))
    gflat = gathered.reshape(-1)
    pg_src = gflat[:N * SRC_L].reshape(N, SRC_L)
    g_t = gflat[N * SRC_L:NIDX]

    bb = bb3d.reshape(N)
    rows_iv = jnp.arange(N, dtype=jnp.int32) * V
    k0 = jnp.minimum((rows_iv + bb * BLK) // WROW, K0_MAX)
    widx = (k0[:, None] + jnp.arange(WIN_ROWS, dtype=jnp.int32)).reshape(-1)
    win = _run_win_gather(flat_pgen, widx).reshape(N, WIN_W)
    cb3d = (k0 * WROW - rows_iv).reshape(NRB, 1, RB1)
    a03d = _run_stage3(win, cb3d, m03d)

    loss2d, pred3d, nw2d, nc2d = _run_stage4(
        p_copy, src, pg_src,
        p_switch.reshape(NRB, 1, RB1), tgt.reshape(NRB, 1, RB1),
        m03d, a03d, g_t.reshape(NRB, 1, RB1))

    return (loss2d[0, 0], pred3d.reshape(N), nw2d[0, 0], nc2d[0, 0])


# speedup vs baseline: 1.0664x; 1.0029x over previous
import functools

import jax
import jax.numpy as jnp
from jax import lax
from jax.experimental import pallas as pl
from jax.experimental.pallas import tpu as pltpu
from jax.experimental.pallas import tpu_sc as plsc

N = 1024
V = 100000
SRC_L = 200
PAD_ID = 0
EPS = 1e-12

RB1 = 128
VB = 8192
BLK = 2048
SUBS = VB // BLK
NRB = N // RB1
NVB = -(-V // VB)
TAIL = V - (NVB - 1) * VB

SC_NC = 2
SC_NS = 16
SC_NW = SC_NC * SC_NS
SC_CHUNK = 128
SC_CPW = 56
SC_GROUP = 7
SC_TOTAL = SC_NW * SC_CPW * SC_CHUNK
NIDX = N * SRC_L + N

BIGI = 2**30


def _stage1_body(x_ref, m_ref, bb_ref):
    vb = pl.program_id(1)

    @pl.when(vb == 0)
    def _init():
        m_ref[0, 0, :] = jnp.full((RB1,), -1.0, jnp.float32)
        bb_ref[0, 0, :] = jnp.zeros((RB1,), jnp.int32)

    def merge(x):
        for sub in range(SUBS):
            bm = jnp.max(x[:, sub * BLK:(sub + 1) * BLK], axis=1)
            m_old = m_ref[0, 0, :]
            better = bm > m_old
            m_ref[0, 0, :] = jnp.where(better, bm, m_old)
            bb_ref[0, 0, :] = jnp.where(
                better, vb * SUBS + sub, bb_ref[0, 0, :])

    @pl.when(vb < NVB - 1)
    def _full():
        merge(x_ref[...])

    @pl.when(vb == NVB - 1)
    def _tail():
        col = lax.broadcasted_iota(jnp.int32, (RB1, VB), 1)
        merge(jnp.where(col < TAIL, x_ref[...], -1.0))


def _run_stage1(p_gen):
    return pl.pallas_call(
        _stage1_body,
        grid=(NRB, NVB),
        in_specs=[pl.BlockSpec((RB1, VB), lambda rb, vb: (rb, vb))],
        out_specs=[
            pl.BlockSpec((1, 1, RB1), lambda rb, vb: (rb, 0, 0)),
            pl.BlockSpec((1, 1, RB1), lambda rb, vb: (rb, 0, 0)),
        ],
        out_shape=[
            jax.ShapeDtypeStruct((NRB, 1, RB1), jnp.float32),
            jax.ShapeDtypeStruct((NRB, 1, RB1), jnp.int32),
        ],
        compiler_params=pltpu.CompilerParams(
            dimension_semantics=(pltpu.PARALLEL, pltpu.ARBITRARY)),
    )(p_gen)


def _sc_gather_body(flat_hbm, idx_hbm, out_hbm, idx_v, rows_v, sem):
    wid = lax.axis_index("s") * SC_NC + lax.axis_index("c")
    base = wid * SC_CPW
    pltpu.sync_copy(idx_hbm.at[pl.ds(base, SC_CPW)], idx_v)

    def fire(g, _):
        for j in range(SC_GROUP):
            c = g * SC_GROUP + j
            pltpu.make_async_copy(
                flat_hbm.at[idx_v.at[c]], rows_v.at[c], sem).start()
        return 0

    def drain(g, _):
        for j in range(SC_GROUP):
            c = g * SC_GROUP + j
            pltpu.make_async_copy(
                flat_hbm.at[idx_v.at[c]], rows_v.at[c], sem).wait()
        return 0

    lax.fori_loop(0, SC_CPW // SC_GROUP, fire, 0)
    lax.fori_loop(0, SC_CPW // SC_GROUP, drain, 0)
    pltpu.sync_copy(rows_v, out_hbm.at[pl.ds(base, SC_CPW)])


def _run_sc_gather(flat_pgen, idx2d):
    mesh = plsc.VectorSubcoreMesh(core_axis_name="c", subcore_axis_name="s")
    fn = functools.partial(
        pl.kernel,
        out_type=jax.ShapeDtypeStruct((SC_NW * SC_CPW, SC_CHUNK), jnp.float32),
        mesh=mesh,
        scratch_types=[
            pltpu.VMEM((SC_CPW, SC_CHUNK), jnp.int32),
            pltpu.VMEM((SC_CPW, SC_CHUNK), jnp.float32),
            pltpu.SemaphoreType.DMA,
        ],
    )(_sc_gather_body)
    return fn(flat_pgen, idx2d)


WROW = 512
WIN_ROWS = 5
WIN_W = WROW * WIN_ROWS
NTROW = N * V // WROW
K0_MAX = NTROW - WIN_ROWS
WG_PW = N * WIN_ROWS // SC_NW
WG_HALF = WG_PW // 2


def _win_gather_body(tab_hbm, idx_hbm, out_hbm, idx_v, rows_v, sem):
    wid = lax.axis_index("s") * SC_NC + lax.axis_index("c")
    base = wid * WG_PW
    pltpu.sync_copy(idx_hbm.at[pl.ds(base, WG_PW)], idx_v)
    c0 = pltpu.make_async_copy(
        tab_hbm.at[idx_v.at[pl.ds(0, WG_HALF)]],
        rows_v.at[pl.ds(0, WG_HALF)], sem)
    c0.start()
    c1 = pltpu.make_async_copy(
        tab_hbm.at[idx_v.at[pl.ds(WG_HALF, WG_HALF)]],
        rows_v.at[pl.ds(WG_HALF, WG_HALF)], sem)
    c1.start()
    c0.wait()
    c1.wait()
    pltpu.sync_copy(rows_v, out_hbm.at[pl.ds(base, WG_PW)])


def _run_win_gather(flat_pgen, widx):
    mesh = plsc.VectorSubcoreMesh(core_axis_name="c", subcore_axis_name="s")
    fn = functools.partial(
        pl.kernel,
        out_type=jax.ShapeDtypeStruct((N * WIN_ROWS, WROW), jnp.float32),
        mesh=mesh,
        scratch_types=[
            pltpu.VMEM((WG_PW,), jnp.int32),
            pltpu.VMEM((WG_PW, WROW), jnp.float32),
            pltpu.SemaphoreType.DMA,
        ],
    )(_win_gather_body)
    return fn(flat_pgen.reshape(NTROW, WROW), widx)


DUP_CH = 40


def _stage4_body(pc_ref, src_ref, pgs_ref, s_ref, tgt_ref, m0_ref, win_ref,
                 cb_ref, gt_ref, loss_ref, pred_ref, nw_ref, nc_ref):
    step = pl.program_id(0)
    pc = pc_ref[...]
    src = src_ref[...]
    pgs = pgs_ref[...]
    s = s_ref[0, 0, :]
    tgt = tgt_ref[0, 0, :]
    m0 = m0_ref[0, 0, :]
    g_t = gt_ref[0, 0, :]

    win = win_ref[...]
    cb = cb_ref[0, 0, :]
    col = cb[:, None] + lax.broadcasted_iota(jnp.int32, (RB1, WIN_W), 1)
    hit = jnp.logical_and(
        jnp.logical_and(col >= 0, col < V), win == m0[:, None])
    a0 = jnp.min(jnp.where(hit, col, BIGI), axis=1)

    acc = jnp.zeros((RB1, SRC_L), jnp.float32)
    for c in range(SRC_L // DUP_CH):
        srcc = src[:, c * DUP_CH:(c + 1) * DUP_CH]
        pcc = pc[:, c * DUP_CH:(c + 1) * DUP_CH]
        eq = srcc[:, :, None] == src[:, None, :]
        acc = acc + jnp.sum(jnp.where(eq, pcc[:, :, None], 0.0), axis=1)

    one_m_s = 1.0 - s
    adj = one_m_s[:, None] * pgs + s[:, None] * acc
    base = one_m_s * m0

    adjmax = jnp.max(adj, axis=1)
    eqa = adj == adjmax[:, None]
    adjidx = jnp.min(jnp.where(eqa, src, BIGI), axis=1)
    pred = jnp.where(
        adjmax > base, adjidx,
        jnp.where(adjmax == base, jnp.minimum(adjidx, a0), a0))
    pred_ref[0, 0, :] = pred

    copy_t = jnp.sum(jnp.where(src == tgt[:, None], pc, 0.0), axis=1)
    tscore = one_m_s * g_t + s * copy_t
    nonpad = tgt != PAD_ID
    nonpad_f = nonpad.astype(jnp.float32)
    loss_part = jnp.sum(jnp.log(tscore + EPS) * nonpad_f) * (-1.0 / N)
    nw_part = jnp.sum(nonpad.astype(jnp.int32))
    nc_part = jnp.sum(jnp.logical_and(pred == tgt, nonpad).astype(jnp.int32))

    @pl.when(step == 0)
    def _init():
        loss_ref[0, 0] = 0.0
        nw_ref[0, 0] = 0
        nc_ref[0, 0] = 0

    loss_ref[0, 0] += loss_part
    nw_ref[0, 0] += nw_part
    nc_ref[0, 0] += nc_part


def _run_stage4(p_copy, src, pg_src, s3d, tgt3d, m03d, win2d, cb3d, gt3d):
    return pl.pallas_call(
        _stage4_body,
        grid=(NRB,),
        in_specs=[
            pl.BlockSpec((RB1, SRC_L), lambda i: (i, 0)),
            pl.BlockSpec((RB1, SRC_L), lambda i: (i, 0)),
            pl.BlockSpec((RB1, SRC_L), lambda i: (i, 0)),
            pl.BlockSpec((1, 1, RB1), lambda i: (i, 0, 0)),
            pl.BlockSpec((1, 1, RB1), lambda i: (i, 0, 0)),
            pl.BlockSpec((1, 1, RB1), lambda i: (i, 0, 0)),
            pl.BlockSpec((RB1, WIN_W), lambda i: (i, 0)),
            pl.BlockSpec((1, 1, RB1), lambda i: (i, 0, 0)),
            pl.BlockSpec((1, 1, RB1), lambda i: (i, 0, 0)),
        ],
        out_specs=[
            pl.BlockSpec(memory_space=pltpu.SMEM, block_shape=(1, 1),
                         index_map=lambda i: (0, 0)),
            pl.BlockSpec((1, 1, RB1), lambda i: (i, 0, 0)),
            pl.BlockSpec(memory_space=pltpu.SMEM, block_shape=(1, 1),
                         index_map=lambda i: (0, 0)),
            pl.BlockSpec(memory_space=pltpu.SMEM, block_shape=(1, 1),
                         index_map=lambda i: (0, 0)),
        ],
        out_shape=[
            jax.ShapeDtypeStruct((1, 1), jnp.float32),
            jax.ShapeDtypeStruct((NRB, 1, RB1), jnp.int32),
            jax.ShapeDtypeStruct((1, 1), jnp.int32),
            jax.ShapeDtypeStruct((1, 1), jnp.int32),
        ],
    )(p_copy, src, pg_src, s3d, tgt3d, m03d, win2d, cb3d, gt3d)


def kernel(p_gen, p_copy, p_switch, tgt, src):
    tgt = tgt.astype(jnp.int32)
    src = src.astype(jnp.int32)

    m03d, bb3d = _run_stage1(p_gen)

    rows = jnp.arange(N, dtype=jnp.int32) * V
    flat_idx = jnp.concatenate([
        (rows[:, None] + src).reshape(-1),
        rows + tgt,
        jnp.zeros((SC_TOTAL - NIDX,), jnp.int32),
    ])
    flat_pgen = p_gen.reshape(-1)
    gathered = _run_sc_gather(
        flat_pgen, flat_idx.reshape(SC_NW * SC_CPW, SC_CHUNK))
    gflat = gathered.reshape(-1)
    pg_src = gflat[:N * SRC_L].reshape(N, SRC_L)
    g_t = gflat[N * SRC_L:NIDX]

    bb = bb3d.reshape(N)
    rows_iv = jnp.arange(N, dtype=jnp.int32) * V
    k0 = jnp.minimum((rows_iv + bb * BLK) // WROW, K0_MAX)
    widx = (k0[:, None] + jnp.arange(WIN_ROWS, dtype=jnp.int32)).reshape(-1)
    win = _run_win_gather(flat_pgen, widx).reshape(N, WIN_W)
    cb3d = (k0 * WROW - rows_iv).reshape(NRB, 1, RB1)

    loss2d, pred3d, nw2d, nc2d = _run_stage4(
        p_copy, src, pg_src,
        p_switch.reshape(NRB, 1, RB1), tgt.reshape(NRB, 1, RB1),
        m03d, win, cb3d, g_t.reshape(NRB, 1, RB1))

    return (loss2d[0, 0], pred3d.reshape(N), nw2d[0, 0], nc2d[0, 0])


# speedup vs baseline: 1.3980x; 1.3109x over previous
import functools

import jax
import jax.numpy as jnp
from jax import lax
from jax.experimental import pallas as pl
from jax.experimental.pallas import tpu as pltpu
from jax.experimental.pallas import tpu_sc as plsc

N = 1024
V = 100000
SRC_L = 200
PAD_ID = 0
EPS = 1e-12

RB1 = 128
VB = 8192
BLK = 2048
SUBS = VB // BLK
NRB = N // RB1
NVB = -(-V // VB)
TAIL = V - (NVB - 1) * VB

SC_NC = 2
SC_NS = 16
SC_NW = SC_NC * SC_NS
SC_CHUNK = 128
SC_CPW = 56
SC_GROUP = 7
SC_TOTAL = SC_NW * SC_CPW * SC_CHUNK
NIDX = N * SRC_L + N

BIGI = 2**30


def _stage1_body(x_ref, m_ref, a0_ref):
    vb = pl.program_id(1)

    @pl.when(vb == 0)
    def _init():
        m_ref[0, 0, :] = jnp.full((RB1,), -1.0, jnp.float32)
        a0_ref[0, 0, :] = jnp.zeros((RB1,), jnp.int32)

    iota = lax.broadcasted_iota(jnp.int32, (RB1, BLK), 1)

    def merge(x):
        for sub in range(SUBS):
            xs = x[:, sub * BLK:(sub + 1) * BLK]
            bm = jnp.max(xs, axis=1)
            hit = xs == bm[:, None]
            inner = jnp.min(jnp.where(hit, iota, BIGI), axis=1)
            m_old = m_ref[0, 0, :]
            better = bm > m_old
            m_ref[0, 0, :] = jnp.where(better, bm, m_old)
            a0_ref[0, 0, :] = jnp.where(
                better, (vb * SUBS + sub) * BLK + inner, a0_ref[0, 0, :])

    @pl.when(vb < NVB - 1)
    def _full():
        merge(x_ref[...])

    @pl.when(vb == NVB - 1)
    def _tail():
        col = lax.broadcasted_iota(jnp.int32, (RB1, VB), 1)
        merge(jnp.where(col < TAIL, x_ref[...], -1.0))


def _run_stage1(p_gen):
    return pl.pallas_call(
        _stage1_body,
        grid=(NRB, NVB),
        in_specs=[pl.BlockSpec((RB1, VB), lambda rb, vb: (rb, vb))],
        out_specs=[
            pl.BlockSpec((1, 1, RB1), lambda rb, vb: (rb, 0, 0)),
            pl.BlockSpec((1, 1, RB1), lambda rb, vb: (rb, 0, 0)),
        ],
        out_shape=[
            jax.ShapeDtypeStruct((NRB, 1, RB1), jnp.float32),
            jax.ShapeDtypeStruct((NRB, 1, RB1), jnp.int32),
        ],
        compiler_params=pltpu.CompilerParams(
            dimension_semantics=(pltpu.PARALLEL, pltpu.ARBITRARY)),
    )(p_gen)


def _sc_gather_body(flat_hbm, idx_hbm, out_hbm, idx_v, rows_v, sem):
    wid = lax.axis_index("s") * SC_NC + lax.axis_index("c")
    base = wid * SC_CPW
    pltpu.sync_copy(idx_hbm.at[pl.ds(base, SC_CPW)], idx_v)

    def fire(g, _):
        for j in range(SC_GROUP):
            c = g * SC_GROUP + j
            pltpu.make_async_copy(
                flat_hbm.at[idx_v.at[c]], rows_v.at[c], sem).start()
        return 0

    def drain(g, _):
        for j in range(SC_GROUP):
            c = g * SC_GROUP + j
            pltpu.make_async_copy(
                flat_hbm.at[idx_v.at[c]], rows_v.at[c], sem).wait()
        return 0

    lax.fori_loop(0, SC_CPW // SC_GROUP, fire, 0)
    lax.fori_loop(0, SC_CPW // SC_GROUP, drain, 0)
    pltpu.sync_copy(rows_v, out_hbm.at[pl.ds(base, SC_CPW)])


def _run_sc_gather(flat_pgen, idx2d):
    mesh = plsc.VectorSubcoreMesh(core_axis_name="c", subcore_axis_name="s")
    fn = functools.partial(
        pl.kernel,
        out_type=jax.ShapeDtypeStruct((SC_NW * SC_CPW, SC_CHUNK), jnp.float32),
        mesh=mesh,
        scratch_types=[
            pltpu.VMEM((SC_CPW, SC_CHUNK), jnp.int32),
            pltpu.VMEM((SC_CPW, SC_CHUNK), jnp.float32),
            pltpu.SemaphoreType.DMA,
        ],
    )(_sc_gather_body)
    return fn(flat_pgen, idx2d)


DUP_CH = 40


def _stage4_body(pc_ref, src_ref, pgs_ref, s_ref, tgt_ref, m0_ref, a0_ref,
                 gt_ref, loss_ref, pred_ref, nw_ref, nc_ref):
    step = pl.program_id(0)
    pc = pc_ref[...]
    src = src_ref[...]
    pgs = pgs_ref[...]
    s = s_ref[0, 0, :]
    tgt = tgt_ref[0, 0, :]
    m0 = m0_ref[0, 0, :]
    a0 = a0_ref[0, 0, :]
    g_t = gt_ref[0, 0, :]

    acc = jnp.zeros((RB1, SRC_L), jnp.float32)
    for c in range(SRC_L // DUP_CH):
        srcc = src[:, c * DUP_CH:(c + 1) * DUP_CH]
        pcc = pc[:, c * DUP_CH:(c + 1) * DUP_CH]
        eq = srcc[:, :, None] == src[:, None, :]
        acc = acc + jnp.sum(jnp.where(eq, pcc[:, :, None], 0.0), axis=1)

    one_m_s = 1.0 - s
    adj = one_m_s[:, None] * pgs + s[:, None] * acc
    base = one_m_s * m0

    adjmax = jnp.max(adj, axis=1)
    eqa = adj == adjmax[:, None]
    adjidx = jnp.min(jnp.where(eqa, src, BIGI), axis=1)
    pred = jnp.where(
        adjmax > base, adjidx,
        jnp.where(adjmax == base, jnp.minimum(adjidx, a0), a0))
    pred_ref[0, 0, :] = pred

    copy_t = jnp.sum(jnp.where(src == tgt[:, None], pc, 0.0), axis=1)
    tscore = one_m_s * g_t + s * copy_t
    nonpad = tgt != PAD_ID
    nonpad_f = nonpad.astype(jnp.float32)
    loss_part = jnp.sum(jnp.log(tscore + EPS) * nonpad_f) * (-1.0 / N)
    nw_part = jnp.sum(nonpad.astype(jnp.int32))
    nc_part = jnp.sum(jnp.logical_and(pred == tgt, nonpad).astype(jnp.int32))

    @pl.when(step == 0)
    def _init():
        loss_ref[0, 0] = 0.0
        nw_ref[0, 0] = 0
        nc_ref[0, 0] = 0

    loss_ref[0, 0] += loss_part
    nw_ref[0, 0] += nw_part
    nc_ref[0, 0] += nc_part


def _run_stage4(p_copy, src, pg_src, s3d, tgt3d, m03d, a03d, gt3d):
    return pl.pallas_call(
        _stage4_body,
        grid=(NRB,),
        in_specs=[
            pl.BlockSpec((RB1, SRC_L), lambda i: (i, 0)),
            pl.BlockSpec((RB1, SRC_L), lambda i: (i, 0)),
            pl.BlockSpec((RB1, SRC_L), lambda i: (i, 0)),
            pl.BlockSpec((1, 1, RB1), lambda i: (i, 0, 0)),
            pl.BlockSpec((1, 1, RB1), lambda i: (i, 0, 0)),
            pl.BlockSpec((1, 1, RB1), lambda i: (i, 0, 0)),
            pl.BlockSpec((1, 1, RB1), lambda i: (i, 0, 0)),
            pl.BlockSpec((1, 1, RB1), lambda i: (i, 0, 0)),
        ],
        out_specs=[
            pl.BlockSpec(memory_space=pltpu.SMEM, block_shape=(1, 1),
                         index_map=lambda i: (0, 0)),
            pl.BlockSpec((1, 1, RB1), lambda i: (i, 0, 0)),
            pl.BlockSpec(memory_space=pltpu.SMEM, block_shape=(1, 1),
                         index_map=lambda i: (0, 0)),
            pl.BlockSpec(memory_space=pltpu.SMEM, block_shape=(1, 1),
                         index_map=lambda i: (0, 0)),
        ],
        out_shape=[
            jax.ShapeDtypeStruct((1, 1), jnp.float32),
            jax.ShapeDtypeStruct((NRB, 1, RB1), jnp.int32),
            jax.ShapeDtypeStruct((1, 1), jnp.int32),
            jax.ShapeDtypeStruct((1, 1), jnp.int32),
        ],
    )(p_copy, src, pg_src, s3d, tgt3d, m03d, a03d, gt3d)


def kernel(p_gen, p_copy, p_switch, tgt, src):
    tgt = tgt.astype(jnp.int32)
    src = src.astype(jnp.int32)

    m03d, a03d = _run_stage1(p_gen)

    rows = jnp.arange(N, dtype=jnp.int32) * V
    flat_idx = jnp.concatenate([
        (rows[:, None] + src).reshape(-1),
        rows + tgt,
        jnp.zeros((SC_TOTAL - NIDX,), jnp.int32),
    ])
    flat_pgen = p_gen.reshape(-1)
    gathered = _run_sc_gather(
        flat_pgen, flat_idx.reshape(SC_NW * SC_CPW, SC_CHUNK))
    gflat = gathered.reshape(-1)
    pg_src = gflat[:N * SRC_L].reshape(N, SRC_L)
    g_t = gflat[N * SRC_L:NIDX]

    loss2d, pred3d, nw2d, nc2d = _run_stage4(
        p_copy, src, pg_src,
        p_switch.reshape(NRB, 1, RB1), tgt.reshape(NRB, 1, RB1),
        m03d, a03d, g_t.reshape(NRB, 1, RB1))

    return (loss2d[0, 0], pred3d.reshape(N), nw2d[0, 0], nc2d[0, 0])


# speedup vs baseline: 1.4023x; 1.0031x over previous
import functools

import jax
import jax.numpy as jnp
from jax import lax
from jax.experimental import pallas as pl
from jax.experimental.pallas import tpu as pltpu
from jax.experimental.pallas import tpu_sc as plsc

N = 1024
V = 100000
SRC_L = 200
PAD_ID = 0
EPS = 1e-12

RB1 = 128
VB = 16384
BLK = 2048
SUBS = VB // BLK
NRB = N // RB1
NVB = -(-V // VB)
TAIL = V - (NVB - 1) * VB

SC_NC = 2
SC_NS = 16
SC_NW = SC_NC * SC_NS
SC_CHUNK = 128
SC_CPW = 56
SC_GROUP = 7
SC_TOTAL = SC_NW * SC_CPW * SC_CHUNK
NIDX = N * SRC_L + N

BIGI = 2**30


def _stage1_body(x_ref, m_ref, a0_ref):
    vb = pl.program_id(1)

    @pl.when(vb == 0)
    def _init():
        m_ref[0, 0, :] = jnp.full((RB1,), -1.0, jnp.float32)
        a0_ref[0, 0, :] = jnp.zeros((RB1,), jnp.int32)

    iota = lax.broadcasted_iota(jnp.int32, (RB1, BLK), 1)

    def merge(x):
        for sub in range(SUBS):
            xs = x[:, sub * BLK:(sub + 1) * BLK]
            bm = jnp.max(xs, axis=1)
            hit = xs == bm[:, None]
            inner = jnp.min(jnp.where(hit, iota, BIGI), axis=1)
            m_old = m_ref[0, 0, :]
            better = bm > m_old
            m_ref[0, 0, :] = jnp.where(better, bm, m_old)
            a0_ref[0, 0, :] = jnp.where(
                better, (vb * SUBS + sub) * BLK + inner, a0_ref[0, 0, :])

    @pl.when(vb < NVB - 1)
    def _full():
        merge(x_ref[...])

    @pl.when(vb == NVB - 1)
    def _tail():
        col = lax.broadcasted_iota(jnp.int32, (RB1, VB), 1)
        merge(jnp.where(col < TAIL, x_ref[...], -1.0))


def _run_stage1(p_gen):
    return pl.pallas_call(
        _stage1_body,
        grid=(NRB, NVB),
        in_specs=[pl.BlockSpec((RB1, VB), lambda rb, vb: (rb, vb))],
        out_specs=[
            pl.BlockSpec((1, 1, RB1), lambda rb, vb: (rb, 0, 0)),
            pl.BlockSpec((1, 1, RB1), lambda rb, vb: (rb, 0, 0)),
        ],
        out_shape=[
            jax.ShapeDtypeStruct((NRB, 1, RB1), jnp.float32),
            jax.ShapeDtypeStruct((NRB, 1, RB1), jnp.int32),
        ],
        compiler_params=pltpu.CompilerParams(
            dimension_semantics=(pltpu.PARALLEL, pltpu.ARBITRARY)),
    )(p_gen)


def _sc_gather_body(flat_hbm, idx_hbm, out_hbm, idx_v, rows_v, sem):
    wid = lax.axis_index("s") * SC_NC + lax.axis_index("c")
    base = wid * SC_CPW
    pltpu.sync_copy(idx_hbm.at[pl.ds(base, SC_CPW)], idx_v)

    def fire(g, _):
        for j in range(SC_GROUP):
            c = g * SC_GROUP + j
            pltpu.make_async_copy(
                flat_hbm.at[idx_v.at[c]], rows_v.at[c], sem).start()
        return 0

    def drain(g, _):
        for j in range(SC_GROUP):
            c = g * SC_GROUP + j
            pltpu.make_async_copy(
                flat_hbm.at[idx_v.at[c]], rows_v.at[c], sem).wait()
        return 0

    lax.fori_loop(0, SC_CPW // SC_GROUP, fire, 0)
    lax.fori_loop(0, SC_CPW // SC_GROUP, drain, 0)
    pltpu.sync_copy(rows_v, out_hbm.at[pl.ds(base, SC_CPW)])


def _run_sc_gather(flat_pgen, idx2d):
    mesh = plsc.VectorSubcoreMesh(core_axis_name="c", subcore_axis_name="s")
    fn = functools.partial(
        pl.kernel,
        out_type=jax.ShapeDtypeStruct((SC_NW * SC_CPW, SC_CHUNK), jnp.float32),
        mesh=mesh,
        scratch_types=[
            pltpu.VMEM((SC_CPW, SC_CHUNK), jnp.int32),
            pltpu.VMEM((SC_CPW, SC_CHUNK), jnp.float32),
            pltpu.SemaphoreType.DMA,
        ],
    )(_sc_gather_body)
    return fn(flat_pgen, idx2d)


DUP_CH = 40


def _stage4_body(pc_ref, src_ref, pgs_ref, s_ref, tgt_ref, m0_ref, a0_ref,
                 gt_ref, loss_ref, pred_ref, nw_ref, nc_ref):
    step = pl.program_id(0)
    pc = pc_ref[...]
    src = src_ref[...]
    pgs = pgs_ref[...]
    s = s_ref[0, 0, :]
    tgt = tgt_ref[0, 0, :]
    m0 = m0_ref[0, 0, :]
    a0 = a0_ref[0, 0, :]
    g_t = gt_ref[0, 0, :]

    acc = jnp.zeros((RB1, SRC_L), jnp.float32)
    for c in range(SRC_L // DUP_CH):
        srcc = src[:, c * DUP_CH:(c + 1) * DUP_CH]
        pcc = pc[:, c * DUP_CH:(c + 1) * DUP_CH]
        eq = srcc[:, :, None] == src[:, None, :]
        acc = acc + jnp.sum(jnp.where(eq, pcc[:, :, None], 0.0), axis=1)

    one_m_s = 1.0 - s
    adj = one_m_s[:, None] * pgs + s[:, None] * acc
    base = one_m_s * m0

    adjmax = jnp.max(adj, axis=1)
    eqa = adj == adjmax[:, None]
    adjidx = jnp.min(jnp.where(eqa, src, BIGI), axis=1)
    pred = jnp.where(
        adjmax > base, adjidx,
        jnp.where(adjmax == base, jnp.minimum(adjidx, a0), a0))
    pred_ref[0, 0, :] = pred

    copy_t = jnp.sum(jnp.where(src == tgt[:, None], pc, 0.0), axis=1)
    tscore = one_m_s * g_t + s * copy_t
    nonpad = tgt != PAD_ID
    nonpad_f = nonpad.astype(jnp.float32)
    loss_part = jnp.sum(jnp.log(tscore + EPS) * nonpad_f) * (-1.0 / N)
    nw_part = jnp.sum(nonpad.astype(jnp.int32))
    nc_part = jnp.sum(jnp.logical_and(pred == tgt, nonpad).astype(jnp.int32))

    @pl.when(step == 0)
    def _init():
        loss_ref[0, 0] = 0.0
        nw_ref[0, 0] = 0
        nc_ref[0, 0] = 0

    loss_ref[0, 0] += loss_part
    nw_ref[0, 0] += nw_part
    nc_ref[0, 0] += nc_part


def _run_stage4(p_copy, src, pg_src, s3d, tgt3d, m03d, a03d, gt3d):
    return pl.pallas_call(
        _stage4_body,
        grid=(NRB,),
        in_specs=[
            pl.BlockSpec((RB1, SRC_L), lambda i: (i, 0)),
            pl.BlockSpec((RB1, SRC_L), lambda i: (i, 0)),
            pl.BlockSpec((RB1, SRC_L), lambda i: (i, 0)),
            pl.BlockSpec((1, 1, RB1), lambda i: (i, 0, 0)),
            pl.BlockSpec((1, 1, RB1), lambda i: (i, 0, 0)),
            pl.BlockSpec((1, 1, RB1), lambda i: (i, 0, 0)),
            pl.BlockSpec((1, 1, RB1), lambda i: (i, 0, 0)),
            pl.BlockSpec((1, 1, RB1), lambda i: (i, 0, 0)),
        ],
        out_specs=[
            pl.BlockSpec(memory_space=pltpu.SMEM, block_shape=(1, 1),
                         index_map=lambda i: (0, 0)),
            pl.BlockSpec((1, 1, RB1), lambda i: (i, 0, 0)),
            pl.BlockSpec(memory_space=pltpu.SMEM, block_shape=(1, 1),
                         index_map=lambda i: (0, 0)),
            pl.BlockSpec(memory_space=pltpu.SMEM, block_shape=(1, 1),
                         index_map=lambda i: (0, 0)),
        ],
        out_shape=[
            jax.ShapeDtypeStruct((1, 1), jnp.float32),
            jax.ShapeDtypeStruct((NRB, 1, RB1), jnp.int32),
            jax.ShapeDtypeStruct((1, 1), jnp.int32),
            jax.ShapeDtypeStruct((1, 1), jnp.int32),
        ],
    )(p_copy, src, pg_src, s3d, tgt3d, m03d, a03d, gt3d)


def kernel(p_gen, p_copy, p_switch, tgt, src):
    tgt = tgt.astype(jnp.int32)
    src = src.astype(jnp.int32)

    m03d, a03d = _run_stage1(p_gen)

    rows = jnp.arange(N, dtype=jnp.int32) * V
    flat_idx = jnp.concatenate([
        (rows[:, None] + src).reshape(-1),
        rows + tgt,
        jnp.zeros((SC_TOTAL - NIDX,), jnp.int32),
    ])
    flat_pgen = p_gen.reshape(-1)
    gathered = _run_sc_gather(
        flat_pgen, flat_idx.reshape(SC_NW * SC_CPW, SC_CHUNK))
    gflat = gathered.reshape(-1)
    pg_src = gflat[:N * SRC_L].reshape(N, SRC_L)
    g_t = gflat[N * SRC_L:NIDX]

    loss2d, pred3d, nw2d, nc2d = _run_stage4(
        p_copy, src, pg_src,
        p_switch.reshape(NRB, 1, RB1), tgt.reshape(NRB, 1, RB1),
        m03d, a03d, g_t.reshape(NRB, 1, RB1))

    return (loss2d[0, 0], pred3d.reshape(N), nw2d[0, 0], nc2d[0, 0])
